# baseline (speedup 1.0000x reference)
import functools

import jax
import jax.numpy as jnp
from jax import lax
from jax.experimental import pallas as pl
from jax.experimental.pallas import tpu as pltpu
from jax.experimental.pallas import tpu_sc as plsc

N_ITEMS = 1000
DIM = 10
N_IN = 5
BATCH = 16384
HIST = 50
P = N_IN * HIST

NC = 2
NS = 16
L = 16
NW = NC * NS
BPW = BATCH // NW
SPITCH = 1024
HC = HIST // 2
SWORDS = P * SPITCH
SSTRIPE = SWORDS // NS


def _s_body(wr_ref, et_ref, b_ref, s_ref):
    s_ref[...] = (
        jnp.dot(wr_ref[...], et_ref[...], preferred_element_type=jnp.float32)
        + b_ref[0, 0] * (1.0 / P)
    )


def _compute_s(wr, et, bias):
    return pl.pallas_call(
        _s_body,
        out_shape=jax.ShapeDtypeStruct((P, SPITCH), jnp.float32),
    )(wr, et, bias)


@functools.partial(
    pl.kernel,
    out_type=jax.ShapeDtypeStruct((BATCH,), jnp.float32),
    mesh=plsc.VectorSubcoreMesh(core_axis_name="c", subcore_axis_name="s"),
    compiler_params=pltpu.CompilerParams(needs_layout_passes=False),
    scratch_types=[
        pltpu.VMEM((HC * SPITCH,), jnp.float32),
        pltpu.VMEM((HIST, BPW), jnp.int32),
        pltpu.VMEM((HIST, BPW), jnp.int32),
        pltpu.VMEM((BPW,), jnp.float32),
        pltpu.VMEM_SHARED((SWORDS,), jnp.float32),
        pltpu.SemaphoreType.DMA,
        pltpu.SemaphoreType.DMA,
    ],
)
def _sc_gather_sum(i0, i1, i2, i3, i4, s_hbm, out_hbm,
                   sbuf, ib0, ib1, acc, shared_s, sem0, sem1):
    cid = lax.axis_index("c")
    sid = lax.axis_index("s")
    wid = sid * NC + cid
    base = wid * BPW
    iota = lax.iota(jnp.int32, L)
    idx_refs = [i0, i1, i2, i3, i4]
    ibufs = [ib0, ib1]
    sems = [sem0, sem1]

    def start(i):
        return pltpu.async_copy(
            idx_refs[i].at[:, pl.ds(base, BPW)], ibufs[i % 2], sems[i % 2])

    handle = start(0)

    pltpu.sync_copy(s_hbm.at[pl.ds(sid * SSTRIPE, SSTRIPE)],
                    shared_s.at[pl.ds(sid * SSTRIPE, SSTRIPE)])
    plsc.subcore_barrier()

    for i in range(N_IN):
        nxt = start(i + 1) if i + 1 < N_IN else None
        handle.wait()
        ibuf = ibufs[i % 2]
        for half in range(2):
            pltpu.sync_copy(
                shared_s.at[pl.ds((i * HIST + half * HC) * SPITCH,
                                  HC * SPITCH)], sbuf)

            def g_body(g, _, ibuf=ibuf, hoff=half * HC,
                       first=(i == 0 and half == 0)):
                accv = jnp.zeros((L,), jnp.float32)
                for h in range(HC):
                    iv = ibuf[hoff + h, pl.ds(g * L, L)]
                    accv = accv + plsc.load_gather(sbuf, [iv + h * SPITCH])
                sl = pl.ds(g * L, L)
                if first:
                    acc[sl] = accv
                else:
                    acc[sl] = acc[sl] + accv
                return _

            lax.fori_loop(0, BPW // L, g_body, None)
        handle = nxt
    pltpu.sync_copy(acc, out_hbm.at[pl.ds(base, BPW)])


def kernel(inputs_0, inputs_1, inputs_2, inputs_3, inputs_4,
           embed_table, dense_W, dense_b):
    wr = dense_W.reshape(P, DIM)
    et = jnp.pad(embed_table.T, ((0, 0), (0, SPITCH - N_ITEMS)))
    bias = dense_b.reshape(1, 1)
    s = _compute_s(wr, et, bias).reshape(SWORDS)
    out = _sc_gather_sum(inputs_0.T, inputs_1.T, inputs_2.T,
                         inputs_3.T, inputs_4.T, s)
    return out.reshape(BATCH, 1)


# speedup vs baseline: 1.0314x; 1.0314x over previous
import functools

import jax
import jax.numpy as jnp
from jax import lax
from jax.experimental import pallas as pl
from jax.experimental.pallas import tpu as pltpu
from jax.experimental.pallas import tpu_sc as plsc

N_ITEMS = 1000
DIM = 10
N_IN = 5
BATCH = 16384
HIST = 50
P = N_IN * HIST

NC = 2
NS = 16
L = 16
NW = NC * NS
BPW = BATCH // NW
SPITCH = 1024
HC = HIST // 2
SWORDS = P * SPITCH
SSTRIPE = SWORDS // NS


def _s_body(wr_ref, et_ref, b_ref, s_ref):
    s_ref[...] = (
        jnp.dot(wr_ref[...], et_ref[...], preferred_element_type=jnp.float32)
        + b_ref[0, 0] * (1.0 / P)
    )


def _compute_s(wr, et, bias):
    return pl.pallas_call(
        _s_body,
        out_shape=jax.ShapeDtypeStruct((P, SPITCH), jnp.float32),
    )(wr, et, bias)


@functools.partial(
    pl.kernel,
    out_type=jax.ShapeDtypeStruct((BATCH,), jnp.float32),
    mesh=plsc.VectorSubcoreMesh(core_axis_name="c", subcore_axis_name="s"),
    compiler_params=pltpu.CompilerParams(needs_layout_passes=False),
    scratch_types=[
        pltpu.VMEM((HC * SPITCH,), jnp.float32),
        pltpu.VMEM((HIST, BPW), jnp.int32),
        pltpu.VMEM((HIST, BPW), jnp.int32),
        pltpu.VMEM((BPW,), jnp.float32),
        pltpu.VMEM_SHARED((SWORDS,), jnp.float32),
        pltpu.SemaphoreType.DMA,
        pltpu.SemaphoreType.DMA,
    ],
)
def _sc_gather_sum(i0, i1, i2, i3, i4, s_hbm, out_hbm,
                   sbuf, ib0, ib1, acc, shared_s, sem0, sem1):
    cid = lax.axis_index("c")
    sid = lax.axis_index("s")
    wid = sid * NC + cid
    base = wid * BPW
    iota = lax.iota(jnp.int32, L)
    idx_refs = [i0, i1, i2, i3, i4]
    ibufs = [ib0, ib1]
    sems = [sem0, sem1]

    def start(i):
        return pltpu.async_copy(
            idx_refs[i].at[:, pl.ds(base, BPW)], ibufs[i % 2], sems[i % 2])

    handle = start(0)

    pltpu.sync_copy(s_hbm.at[pl.ds(sid * SSTRIPE, SSTRIPE)],
                    shared_s.at[pl.ds(sid * SSTRIPE, SSTRIPE)])
    plsc.subcore_barrier()

    for i in range(N_IN):
        nxt = start(i + 1) if i + 1 < N_IN else None
        handle.wait()
        ibuf = ibufs[i % 2]
        for half in range(2):
            pltpu.sync_copy(
                shared_s.at[pl.ds((i * HIST + half * HC) * SPITCH,
                                  HC * SPITCH)], sbuf)

            def g_body(g, _, ibuf=ibuf, hoff=half * HC,
                       first=(i == 0 and half == 0)):
                acc0 = jnp.zeros((L,), jnp.float32)
                acc1 = jnp.zeros((L,), jnp.float32)
                for h in range(HC):
                    iv = ibuf[hoff + h, pl.ds(g * L, L)]
                    g_v = plsc.load_gather(sbuf, [iv + h * SPITCH])
                    if h % 2 == 0:
                        acc0 = acc0 + g_v
                    else:
                        acc1 = acc1 + g_v
                accv = acc0 + acc1
                sl = pl.ds(g * L, L)
                if first:
                    acc[sl] = accv
                else:
                    acc[sl] = acc[sl] + accv
                return _

            lax.fori_loop(0, BPW // L, g_body, None)
        handle = nxt
    pltpu.sync_copy(acc, out_hbm.at[pl.ds(base, BPW)])


def kernel(inputs_0, inputs_1, inputs_2, inputs_3, inputs_4,
           embed_table, dense_W, dense_b):
    wr = dense_W.reshape(P, DIM)
    et = jnp.pad(embed_table.T, ((0, 0), (0, SPITCH - N_ITEMS)))
    bias = dense_b.reshape(1, 1)
    s = _compute_s(wr, et, bias).reshape(SWORDS)
    out = _sc_gather_sum(inputs_0.T, inputs_1.T, inputs_2.T,
                         inputs_3.T, inputs_4.T, s)
    return out.reshape(BATCH, 1)
